# Initial kernel scaffold; baseline (speedup 1.0000x reference)
import functools

import jax
import jax.numpy as jnp
from jax import lax
from jax.experimental import pallas as pl
from jax.experimental.pallas import tpu as pltpu
from jax.experimental.pallas import tpu_sc as plsc

B = 16384
F = 26
V = 100000
D = 16
H = 256
FD = F * D

NC = 2
NS = 16
L = 16
NW = NC * NS
BPW = B // NW
IPW = BPW * F
CHUNK = 128
NCH = IPW // CHUNK

_mesh = plsc.VectorSubcoreMesh(
    core_axis_name="c", subcore_axis_name="s", num_cores=NC, num_subcores=NS
)


@functools.partial(
    pl.kernel,
    out_type=(
        jax.ShapeDtypeStruct((B * F, D), jnp.float32),
        jax.ShapeDtypeStruct((B,), jnp.float32),
    ),
    mesh=_mesh,
    scratch_types=(
        pltpu.VMEM((NCH, CHUNK), jnp.int32),
        pltpu.VMEM((CHUNK, D), jnp.float32),
        pltpu.VMEM((IPW, 1), jnp.float32),
        pltpu.VMEM((BPW,), jnp.float32),
        pltpu.SemaphoreType.DMA,
        pltpu.SemaphoreType.DMA,
    ),
)
def _sc_gather(idx_hbm, emb_hbm, widew_hbm, embout_hbm, wideout_hbm,
               idx_v, rows_v, wrows_v, wacc_v, sem_g, sem_w):
    wid = lax.axis_index("s") * NC + lax.axis_index("c")

    pltpu.sync_copy(idx_hbm.at[pl.ds(wid * NCH, NCH)], idx_v)

    groups_per_row = CHUNK // L

    def _fix(i, carry):
        r = i // groups_per_row
        c = (i % groups_per_row) * L
        pos = i * L + lax.iota(jnp.int32, (L,))
        f = lax.rem(pos, F)
        idx_v[r, pl.ds(c, L)] = idx_v[r, pl.ds(c, L)] + f * V
        return carry

    lax.fori_loop(0, NCH * groups_per_row, _fix, 0)

    def _chunk(s, carry):
        pltpu.async_copy(
            widew_hbm.at[idx_v.at[s]], wrows_v.at[pl.ds(s * CHUNK, CHUNK)], sem_w
        )
        pltpu.async_copy(emb_hbm.at[idx_v.at[s]], rows_v, sem_g).wait()
        pltpu.sync_copy(
            rows_v, embout_hbm.at[pl.ds((wid * NCH + s) * CHUNK, CHUNK)]
        )
        return carry

    lax.fori_loop(0, NCH, _chunk, 0)

    pltpu.make_async_copy(
        widew_hbm.at[pl.ds(0, IPW)], wrows_v, sem_w
    ).wait()

    def _wsum(j, carry):
        base = j * L
        lanes = base + lax.iota(jnp.int32, (L,))
        zeros = jnp.zeros((L,), jnp.int32)
        acc = jnp.zeros((L,), jnp.float32)
        for k in range(F):
            acc = acc + plsc.load_gather(wrows_v, [lanes * F + k, zeros])
        wacc_v[pl.ds(base, L)] = acc
        return carry

    lax.fori_loop(0, BPW // L, _wsum, 0)

    pltpu.sync_copy(wacc_v, wideout_hbm.at[pl.ds(wid * BPW, BPW)])


_BM = 2048


def _mlp_body(x_ref, wide_ref, w1_ref, b1_ref, w2_ref, b2_ref, wf_ref, bf_ref,
              o_ref):
    x = x_ref[...]
    h = jnp.maximum(jnp.dot(x, w1_ref[...]) + b1_ref[...], 0.0)
    h = jnp.maximum(jnp.dot(h, w2_ref[...]) + b2_ref[...], 0.0)
    d = jnp.dot(h, wf_ref[...]) + bf_ref[...]
    o_ref[...] = jax.nn.sigmoid(0.5 * wide_ref[...] + 0.5 * d)


_mlp = pl.pallas_call(
    _mlp_body,
    grid=(B // _BM,),
    in_specs=[
        pl.BlockSpec((_BM, FD), lambda i: (i, 0)),
        pl.BlockSpec((_BM, 1), lambda i: (i, 0)),
        pl.BlockSpec((FD, H), lambda i: (0, 0)),
        pl.BlockSpec((1, H), lambda i: (0, 0)),
        pl.BlockSpec((H, H), lambda i: (0, 0)),
        pl.BlockSpec((1, H), lambda i: (0, 0)),
        pl.BlockSpec((H, 1), lambda i: (0, 0)),
        pl.BlockSpec((1, 1), lambda i: (0, 0)),
    ],
    out_specs=pl.BlockSpec((_BM, 1), lambda i: (i, 0)),
    out_shape=jax.ShapeDtypeStruct((B, 1), jnp.float32),
)


def kernel(inputs, embed_tables, W1, b1, W2, b2, Wf, bf, wide_w):
    idx = inputs.astype(jnp.int32).reshape(NW * NCH, CHUNK)
    emb_flat = embed_tables.reshape(F * V, D)
    embed_out, wide_out = _sc_gather(idx, emb_flat, wide_w)
    x = embed_out.reshape(B, FD)
    return _mlp(
        x,
        wide_out.reshape(B, 1),
        W1,
        b1.reshape(1, H),
        W2,
        b2.reshape(1, H),
        Wf,
        bf.reshape(1, 1),
    )


# speedup vs baseline: 7.2861x; 7.2861x over previous
import functools

import jax
import jax.numpy as jnp
from jax import lax
from jax.experimental import pallas as pl
from jax.experimental.pallas import tpu as pltpu
from jax.experimental.pallas import tpu_sc as plsc

B = 16384
F = 26
V = 100000
D = 16
H = 256
FD = F * D

NC = 2
NS = 16
L = 16
NW = NC * NS
BPW = B // NW
IPW = BPW * F
CHUNK = 128
NCH = IPW // CHUNK

_mesh = plsc.VectorSubcoreMesh(
    core_axis_name="c", subcore_axis_name="s", num_cores=NC, num_subcores=NS
)


@functools.partial(
    pl.kernel,
    out_type=(
        jax.ShapeDtypeStruct((B * F, D), jnp.float32),
        jax.ShapeDtypeStruct((B * F,), jnp.float32),
    ),
    mesh=_mesh,
    scratch_types=(
        pltpu.VMEM((NCH, CHUNK), jnp.int32),
        pltpu.VMEM((NCH, CHUNK), jnp.int32),
        pltpu.VMEM((CHUNK, D), jnp.float32),
        pltpu.VMEM((CHUNK, L), jnp.float32),
        pltpu.VMEM((CHUNK,), jnp.float32),
        pltpu.SemaphoreType.DMA,
        pltpu.SemaphoreType.DMA,
    ),
    compiler_params=pltpu.CompilerParams(
        use_tc_tiling_on_sc=False, needs_layout_passes=False
    ),
)
def _sc_gather(idx_hbm, emb_hbm, widew_hbm, embout_hbm, wideout_hbm,
               idx_v, mdx_v, rows_v, wrows_v, wsel_v, sem_g, sem_w):
    wid = lax.axis_index("s") * NC + lax.axis_index("c")

    pltpu.sync_copy(idx_hbm.at[pl.ds(wid * NCH, NCH)], idx_v)

    groups_per_row = CHUNK // L

    def _fix(i, carry):
        r = i // groups_per_row
        c = (i % groups_per_row) * L
        pos = i * L + lax.iota(jnp.int32, L)
        f = lax.rem(pos, F)
        flat = idx_v[r, pl.ds(c, L)] + f * V
        idx_v[r, pl.ds(c, L)] = flat
        mdx_v[r, pl.ds(c, L)] = lax.shift_right_logical(flat, 4)
        return carry

    lax.fori_loop(0, NCH * groups_per_row, _fix, 0)

    def _chunk(s, carry):
        wd = pltpu.async_copy(widew_hbm.at[mdx_v.at[s]], wrows_v, sem_w)
        ed = pltpu.async_copy(emb_hbm.at[idx_v.at[s]], rows_v, sem_g)
        ed.wait()
        out_off = (wid * NCH + s) * CHUNK
        pltpu.sync_copy(rows_v, embout_hbm.at[pl.ds(out_off, CHUNK)])
        wd.wait()
        lanes = lax.iota(jnp.int32, L)
        for g in range(groups_per_row):
            m = lax.bitwise_and(idx_v[s, pl.ds(g * L, L)], L - 1)
            wsel_v[pl.ds(g * L, L)] = plsc.load_gather(
                wrows_v, [g * L + lanes, m]
            )
        pltpu.sync_copy(wsel_v, wideout_hbm.at[pl.ds(out_off, CHUNK)])
        return carry

    lax.fori_loop(0, NCH, _chunk, 0)


_BM = 2048


def _mlp_body(x_ref, wide_ref, w1_ref, b1_ref, w2_ref, b2_ref, wf_ref, bf_ref,
              o_ref):
    x = x_ref[...]
    h = jnp.maximum(jnp.dot(x, w1_ref[...]) + b1_ref[...], 0.0)
    h = jnp.maximum(jnp.dot(h, w2_ref[...]) + b2_ref[...], 0.0)
    d = jnp.dot(h, wf_ref[...]) + bf_ref[...]
    wsum = jnp.sum(wide_ref[...], axis=1, keepdims=True)
    o_ref[...] = jax.nn.sigmoid(0.5 * wsum + 0.5 * d)


_mlp = pl.pallas_call(
    _mlp_body,
    grid=(B // _BM,),
    in_specs=[
        pl.BlockSpec((_BM, FD), lambda i: (i, 0)),
        pl.BlockSpec((_BM, F), lambda i: (i, 0)),
        pl.BlockSpec((FD, H), lambda i: (0, 0)),
        pl.BlockSpec((1, H), lambda i: (0, 0)),
        pl.BlockSpec((H, H), lambda i: (0, 0)),
        pl.BlockSpec((1, H), lambda i: (0, 0)),
        pl.BlockSpec((H, 1), lambda i: (0, 0)),
        pl.BlockSpec((1, 1), lambda i: (0, 0)),
    ],
    out_specs=pl.BlockSpec((_BM, 1), lambda i: (i, 0)),
    out_shape=jax.ShapeDtypeStruct((B, 1), jnp.float32),
)


def kernel(inputs, embed_tables, W1, b1, W2, b2, Wf, bf, wide_w):
    idx = inputs.astype(jnp.int32).reshape(NW * NCH, CHUNK)
    emb_flat = embed_tables.reshape(F * V, D)
    wide_rows = wide_w.reshape(F * V // L, L)
    embed_out, wide_out = _sc_gather(idx, emb_flat, wide_rows)
    x = embed_out.reshape(B, FD)
    return _mlp(
        x,
        wide_out.reshape(B, F),
        W1,
        b1.reshape(1, H),
        W2,
        b2.reshape(1, H),
        Wf,
        bf.reshape(1, 1),
    )
